# baseline (speedup 1.0000x reference)
import functools

import jax
import jax.numpy as jnp
from jax import lax
from jax.experimental import pallas as pl
from jax.experimental.pallas import tpu as pltpu
from jax.experimental.pallas import tpu_sc as plsc

_NC = 2
_NS = 16
_NW = _NC * _NS

_CH = 128


def _make_gather(v, d, b_total):
    assert b_total % _NW == 0
    b_per_w = b_total // _NW
    assert b_per_w % _CH == 0
    n_ch = b_per_w // _CH

    mesh = plsc.VectorSubcoreMesh(core_axis_name="c", subcore_axis_name="s")

    @functools.partial(
        pl.kernel,
        mesh=mesh,
        compiler_params=pltpu.CompilerParams(use_tc_tiling_on_sc=False),
        out_type=jax.ShapeDtypeStruct((_NW, n_ch, _CH, d), jnp.float32),
        scratch_types=[
            pltpu.VMEM((n_ch, _CH), jnp.int32),
            pltpu.VMEM((_CH, d), jnp.float32),
            pltpu.SemaphoreType.DMA,
        ],
    )
    def gather_kernel(idx_hbm, table_hbm, out_hbm, idx_v, rows_v, gsem):
        wid = lax.axis_index("s") * _NC + lax.axis_index("c")
        pltpu.sync_copy(idx_hbm.at[wid], idx_v)

        def body(j, _):
            pltpu.async_copy(table_hbm.at[idx_v.at[j]], rows_v, gsem).wait()
            pltpu.sync_copy(rows_v, out_hbm.at[wid, j])
            return ()

        lax.fori_loop(0, n_ch, body, ())

    return gather_kernel


def kernel(stroke_orders, embedding_table):
    batch, hist = stroke_orders.shape
    v, d = embedding_table.shape
    b_total = batch * hist
    idx = stroke_orders.astype(jnp.int32).reshape(_NW, b_total // (_NW * _CH), _CH)
    out = _make_gather(v, d, b_total)(idx, embedding_table)
    return out.reshape(batch, hist, d)


# speedup vs baseline: 1.0474x; 1.0474x over previous
import functools

import jax
import jax.numpy as jnp
from jax import lax
from jax.experimental import pallas as pl
from jax.experimental.pallas import tpu as pltpu
from jax.experimental.pallas import tpu_sc as plsc

_NC = 2
_NS = 16
_NW = _NC * _NS

_CH = 128
_NBUF = 5


def _make_gather(v, d, b_total):
    assert b_total % _NW == 0
    b_per_w = b_total // _NW
    assert b_per_w % _CH == 0
    n_ch = b_per_w // _CH
    assert n_ch % _NBUF == 0 and n_ch // _NBUF >= 2
    n_grp = n_ch // _NBUF

    mesh = plsc.VectorSubcoreMesh(core_axis_name="c", subcore_axis_name="s")

    @functools.partial(
        pl.kernel,
        mesh=mesh,
        compiler_params=pltpu.CompilerParams(use_tc_tiling_on_sc=False),
        out_type=jax.ShapeDtypeStruct((_NW, n_ch, _CH, d), jnp.float32),
        scratch_types=[
            pltpu.VMEM((n_ch, _CH), jnp.int32),
            pltpu.VMEM((_NBUF, _CH, d), jnp.float32),
            pltpu.SemaphoreType.DMA((_NBUF,)),
            pltpu.SemaphoreType.DMA((_NBUF,)),
        ],
    )
    def gather_kernel(idx_hbm, table_hbm, out_hbm, idx_v, rows_v, gsem, ssem):
        wid = lax.axis_index("s") * _NC + lax.axis_index("c")
        pltpu.sync_copy(idx_hbm.at[wid], idx_v)

        for b in range(_NBUF):
            pltpu.async_copy(table_hbm.at[idx_v.at[b]], rows_v.at[b], gsem.at[b])

        def group(g, _):
            for b in range(_NBUF):
                j = g * _NBUF + b
                pltpu.make_async_copy(
                    table_hbm.at[idx_v.at[b]], rows_v.at[b], gsem.at[b]
                ).wait()
                pltpu.async_copy(rows_v.at[b], out_hbm.at[wid, j], ssem.at[b])
                jn = j + _NBUF

                @pl.when(jn < n_ch)
                def _():
                    pltpu.make_async_copy(
                        rows_v.at[b], out_hbm.at[wid, j], ssem.at[b]
                    ).wait()
                    pltpu.async_copy(
                        table_hbm.at[idx_v.at[jn]], rows_v.at[b], gsem.at[b]
                    )

            return ()

        lax.fori_loop(0, n_grp, group, ())

        for b in range(_NBUF):
            pltpu.make_async_copy(
                rows_v.at[b], out_hbm.at[wid, n_ch - _NBUF + b], ssem.at[b]
            ).wait()

    return gather_kernel


def kernel(stroke_orders, embedding_table):
    batch, hist = stroke_orders.shape
    v, d = embedding_table.shape
    b_total = batch * hist
    idx = stroke_orders.astype(jnp.int32).reshape(_NW, b_total // (_NW * _CH), _CH)
    out = _make_gather(v, d, b_total)(idx, embedding_table)
    return out.reshape(batch, hist, d)
